# Initial kernel scaffold; baseline (speedup 1.0000x reference)
import jax
import jax.numpy as jnp
from jax.experimental import pallas as pl


def kernel(x, edge_index, convW, convB, lin_W, lin_b):
    raise NotImplementedError("write your pallas kernel here")


# speedup vs baseline: 5.6845x; 5.6845x over previous
import functools

import jax
import jax.numpy as jnp
from jax import lax
from jax.experimental import pallas as pl
from jax.experimental.pallas import tpu as pltpu
from jax.experimental.pallas import tpu_sc as plsc

N = 10000
D = 256
DH = 128
NLAYERS = 6
NC = 2
NS = 16
CH = 128
E_RAW = 160000 + N
E_PAD = ((E_RAW + NS * CH - 1) // (NS * CH)) * (NS * CH)
N_PAD = 10240
ROWS_T = N_PAD // NS
BN = 1000


def _make_sc_pass(gather: bool):
    mesh = plsc.VectorSubcoreMesh(core_axis_name="c", subcore_axis_name="s")
    nchunk = E_PAD // (NS * CH)

    @functools.partial(
        pl.kernel,
        mesh=mesh,
        out_type=jax.ShapeDtypeStruct((NC, N_PAD, DH), jnp.float32),
        scratch_types=[
            pltpu.VMEM((CH,), jnp.int32),
            pltpu.VMEM((CH,), jnp.int32),
            pltpu.VMEM((CH, DH), jnp.float32),
            pltpu.VMEM((CH, DH), jnp.float32),
            pltpu.VMEM_SHARED((N_PAD, DH), jnp.float32),
            pltpu.SemaphoreType.DMA,
        ],
    )
    def sc_pass(tbl, gidx, ridx, fill, out, gi_v, ri_v, rows_v, buf_v, acc, sem):
        c = lax.axis_index("c")
        s = lax.axis_index("s")
        pltpu.sync_copy(fill.at[0], buf_v)
        for j in range(ROWS_T // CH):
            pltpu.sync_copy(buf_v, acc.at[pl.ds(s * ROWS_T + j * CH, CH)])
        if not gather:
            pltpu.sync_copy(fill.at[1], rows_v)
        plsc.subcore_barrier()

        def body(i, carry):
            base = (i * NS + s) * CH
            pltpu.sync_copy(ridx.at[pl.ds(base, CH)], ri_v)
            if gather:
                pltpu.sync_copy(gidx.at[c, pl.ds(base, CH)], gi_v)
                pltpu.async_copy(tbl.at[gi_v], rows_v, sem).wait()
            pltpu.sync_copy(rows_v, acc.at[ri_v], add=True)
            return carry

        lax.fori_loop(0, nchunk, body, 0)
        plsc.subcore_barrier()
        for j in range(ROWS_T // CH):
            start = s * ROWS_T + j * CH
            pltpu.sync_copy(acc.at[pl.ds(start, CH)], buf_v)
            pltpu.sync_copy(buf_v, out.at[c, pl.ds(start, CH)])

    return sc_pass


_sc_spmm = _make_sc_pass(gather=True)
_sc_deg = _make_sc_pass(gather=False)


def _tc_prologue(deg_ref, x_ref, w0_ref, dis_ref, t0_ref):
    dis = lax.rsqrt(deg_ref[0])
    dis_ref[...] = dis
    d2 = jnp.concatenate([dis, dis], axis=1)
    t = jnp.dot(x_ref[...], w0_ref[...], preferred_element_type=jnp.float32)
    t0_ref[...] = t * d2


def _tc_layer(u_ref, dis_ref, b_ref, w_ref, h_ref, t_ref):
    dis = dis_ref[...]
    d2 = jnp.concatenate([dis, dis], axis=1)
    u = jnp.concatenate([u_ref[0], u_ref[1]], axis=1)
    h = jnp.maximum(u * d2 + b_ref[...], 0.0)
    h_ref[...] = h
    t = jnp.dot(h, w_ref[...], preferred_element_type=jnp.float32)
    t_ref[...] = t * d2


def _tc_epilogue(u_ref, dis_ref, b_ref, h0, h1, h2, h3, h4, lw_ref, lb_ref,
                 out_ref):
    dis = dis_ref[...]
    d2 = jnp.concatenate([dis, dis], axis=1)
    u = jnp.concatenate([u_ref[0], u_ref[1]], axis=1)
    h5 = jnp.maximum(u * d2 + b_ref[...], 0.0)
    lw = lw_ref[...]
    acc = lb_ref[...] + jnp.dot(h5, lw[5 * D:6 * D],
                                preferred_element_type=jnp.float32)
    for i, h in enumerate((h0, h1, h2, h3, h4)):
        acc = acc + jnp.dot(h[...], lw[i * D:(i + 1) * D],
                            preferred_element_type=jnp.float32)
    m = jnp.max(acc, axis=1, keepdims=True)
    e = jnp.exp(acc - m)
    out_ref[...] = acc - m - jnp.log(jnp.sum(e, axis=1, keepdims=True))


def _row_block(d):
    return pl.BlockSpec((BN, d), lambda i: (i, 0))


def _half_block():
    return pl.BlockSpec((NC, BN, DH), lambda i: (0, i, 0))


def _full_block(r, c):
    return pl.BlockSpec((r, c), lambda i: (0, 0))


def kernel(x, edge_index, convW, convB, lin_W, lin_b):
    row = edge_index[1].astype(jnp.int32)
    col = edge_index[0].astype(jnp.int32)
    loop = jnp.arange(N, dtype=jnp.int32)
    row = jnp.concatenate([row, loop])
    col = jnp.concatenate([col, loop])
    npad = E_PAD - E_RAW
    row = jnp.concatenate([row, jnp.full((npad,), N, jnp.int32)])
    col = jnp.concatenate([col, jnp.zeros((npad,), jnp.int32)])
    gidx = jnp.stack([2 * col, 2 * col + 1])
    fill = jnp.stack([jnp.zeros((CH, DH), jnp.float32),
                      jnp.ones((CH, DH), jnp.float32)])
    dummy_tbl = jnp.zeros((2 * N, DH), jnp.float32)

    deg = _sc_deg(dummy_tbl, gidx, row, fill)

    grid = (N // BN,)
    dis, t = pl.pallas_call(
        _tc_prologue,
        grid=grid,
        in_specs=[_half_block(), _row_block(D), _full_block(D, D)],
        out_specs=[_row_block(DH), _row_block(D)],
        out_shape=[jax.ShapeDtypeStruct((N, DH), jnp.float32),
                   jax.ShapeDtypeStruct((N, D), jnp.float32)],
    )(deg, x, convW[0])

    hs = []
    for i in range(NLAYERS):
        u = _sc_spmm(t.reshape(2 * N, DH), gidx, row, fill)
        if i == NLAYERS - 1:
            break
        h, t = pl.pallas_call(
            _tc_layer,
            grid=grid,
            in_specs=[_half_block(), _row_block(DH), _full_block(1, D),
                      _full_block(D, D)],
            out_specs=[_row_block(D), _row_block(D)],
            out_shape=[jax.ShapeDtypeStruct((N, D), jnp.float32),
                       jax.ShapeDtypeStruct((N, D), jnp.float32)],
        )(u, dis, convB[i].reshape(1, D), convW[i + 1])
        hs.append(h)

    out = pl.pallas_call(
        _tc_epilogue,
        grid=grid,
        in_specs=[_half_block(), _row_block(DH), _full_block(1, D)]
        + [_row_block(D)] * 5
        + [_full_block(NLAYERS * D, D), _full_block(1, D)],
        out_specs=_row_block(D),
        out_shape=jax.ShapeDtypeStruct((N, D), jnp.float32),
    )(u, dis, convB[NLAYERS - 1].reshape(1, D), *hs,
      lin_W, lin_b.reshape(1, D))
    return out


# speedup vs baseline: 7.5250x; 1.3238x over previous
import functools

import jax
import jax.numpy as jnp
from jax import lax
from jax.experimental import pallas as pl
from jax.experimental.pallas import tpu as pltpu
from jax.experimental.pallas import tpu_sc as plsc

N = 10000
D = 256
DH = 128
NLAYERS = 6
NC = 2
NS = 16
CH = 128
E_RAW = 160000 + N
E_PAD = ((E_RAW + NS * CH - 1) // (NS * CH)) * (NS * CH)
N_PAD = 10240
ROWS_T = N_PAD // NS
BN = 1000


NB = 2
NSLOT = 2 * NB
NCHUNK = E_PAD // (NS * CH)


def _make_sc_pass(gather: bool):
    mesh = plsc.VectorSubcoreMesh(core_axis_name="c", subcore_axis_name="s")

    @functools.partial(
        pl.kernel,
        mesh=mesh,
        out_type=jax.ShapeDtypeStruct((NC, N_PAD, DH), jnp.float32),
        scratch_types=[pltpu.VMEM((CH,), jnp.int32)] * NSLOT
        + [pltpu.VMEM((CH,), jnp.int32)] * NSLOT
        + [pltpu.VMEM((CH, DH), jnp.float32)] * NB
        + [pltpu.SemaphoreType.DMA] * NSLOT
        + [pltpu.SemaphoreType.DMA] * NB
        + [pltpu.SemaphoreType.DMA] * NB
        + [pltpu.VMEM_SHARED((N_PAD, DH), jnp.float32)],
    )
    def sc_pass(tbl, gidx, ridx, fill, out, *rest):
        gi = rest[:NSLOT]
        ri = rest[NSLOT:2 * NSLOT]
        rows = rest[2 * NSLOT:2 * NSLOT + NB]
        isems = rest[2 * NSLOT + NB:3 * NSLOT + NB]
        gsems = rest[3 * NSLOT + NB:3 * NSLOT + 2 * NB]
        ssems = rest[3 * NSLOT + 2 * NB:3 * NSLOT + 3 * NB]
        acc = rest[-1]
        c = lax.axis_index("c")
        s = lax.axis_index("s")
        nsuper = NCHUNK // (2 * NB)

        def load_idx(chunk, slot):
            pltpu.async_copy(ridx.at[s, chunk], ri[slot], isems[slot])
            if gather:
                pltpu.async_copy(gidx.at[c, s, chunk], gi[slot], isems[slot])

        def wait_idx(slot):
            pltpu.make_async_copy(ridx.at[s, 0], ri[slot],
                                  isems[slot]).wait()
            if gather:
                pltpu.make_async_copy(ridx.at[s, 0], gi[slot],
                                      isems[slot]).wait()

        def drain_scatter(b):
            pltpu.make_async_copy(tbl.at[pl.ds(0, CH)], rows[b],
                                  ssems[b]).wait()

        for k in range(NB):
            load_idx(k, k)
        pltpu.sync_copy(fill.at[0], rows[0])
        for j in range(ROWS_T // CH):
            pltpu.sync_copy(rows[0], acc.at[pl.ds(s * ROWS_T + j * CH, CH)])
        if not gather:
            for b in range(NB):
                pltpu.sync_copy(fill.at[1], rows[b])
        plsc.subcore_barrier()

        def super_(j, carry):
            for p in range(2):
                for b in range(NB):
                    slot = p * NB + b
                    wait_idx(slot)
                    if p == 0:
                        @pl.when(j > 0)
                        def _(b=b):
                            drain_scatter(b)
                    else:
                        drain_scatter(b)
                    if gather:
                        pltpu.async_copy(tbl.at[gi[slot]], rows[b],
                                         gsems[b])
                for b in range(NB):
                    slot = p * NB + b
                    chunk = j * (2 * NB) + p * NB + b
                    if gather:
                        pltpu.make_async_copy(tbl.at[pl.ds(0, CH)], rows[b],
                                              gsems[b]).wait()
                    pltpu.async_copy(rows[b], acc.at[ri[slot]], ssems[b],
                                     add=True)
                    nslot = (slot + NB) % NSLOT
                    if p == 0:
                        load_idx(chunk + NB, nslot)
                    else:
                        @pl.when(j < nsuper - 1)
                        def _(chunk=chunk, nslot=nslot):
                            load_idx(chunk + NB, nslot)
            return carry

        lax.fori_loop(0, nsuper, super_, 0)
        for b in range(NB):
            drain_scatter(b)
        plsc.subcore_barrier()
        for j in range(ROWS_T // CH):
            start = s * ROWS_T + j * CH
            pltpu.sync_copy(acc.at[pl.ds(start, CH)], rows[0])
            pltpu.sync_copy(rows[0], out.at[c, pl.ds(start, CH)])

    return sc_pass


_sc_spmm = _make_sc_pass(gather=True)
_sc_deg = _make_sc_pass(gather=False)


def _tc_prologue(deg_ref, x_ref, w0_ref, dis_ref, t0_ref):
    dis = lax.rsqrt(deg_ref[0])
    dis_ref[...] = dis
    d2 = jnp.concatenate([dis, dis], axis=1)
    t = jnp.dot(x_ref[...], w0_ref[...], preferred_element_type=jnp.float32)
    t0_ref[...] = t * d2


def _tc_layer(u_ref, dis_ref, b_ref, w_ref, h_ref, t_ref):
    dis = dis_ref[...]
    d2 = jnp.concatenate([dis, dis], axis=1)
    u = jnp.concatenate([u_ref[0], u_ref[1]], axis=1)
    h = jnp.maximum(u * d2 + b_ref[...], 0.0)
    h_ref[...] = h
    t = jnp.dot(h, w_ref[...], preferred_element_type=jnp.float32)
    t_ref[...] = t * d2


def _tc_epilogue(u_ref, dis_ref, b_ref, h0, h1, h2, h3, h4, lw_ref, lb_ref,
                 out_ref):
    dis = dis_ref[...]
    d2 = jnp.concatenate([dis, dis], axis=1)
    u = jnp.concatenate([u_ref[0], u_ref[1]], axis=1)
    h5 = jnp.maximum(u * d2 + b_ref[...], 0.0)
    lw = lw_ref[...]
    acc = lb_ref[...] + jnp.dot(h5, lw[5 * D:6 * D],
                                preferred_element_type=jnp.float32)
    for i, h in enumerate((h0, h1, h2, h3, h4)):
        acc = acc + jnp.dot(h[...], lw[i * D:(i + 1) * D],
                            preferred_element_type=jnp.float32)
    m = jnp.max(acc, axis=1, keepdims=True)
    e = jnp.exp(acc - m)
    out_ref[...] = acc - m - jnp.log(jnp.sum(e, axis=1, keepdims=True))


def _row_block(d):
    return pl.BlockSpec((BN, d), lambda i: (i, 0))


def _half_block():
    return pl.BlockSpec((NC, BN, DH), lambda i: (0, i, 0))


def _full_block(r, c):
    return pl.BlockSpec((r, c), lambda i: (0, 0))


def kernel(x, edge_index, convW, convB, lin_W, lin_b):
    row = edge_index[1].astype(jnp.int32)
    col = edge_index[0].astype(jnp.int32)
    loop = jnp.arange(N, dtype=jnp.int32)
    row = jnp.concatenate([row, loop])
    col = jnp.concatenate([col, loop])
    npad = E_PAD - E_RAW
    row = jnp.concatenate([row, jnp.full((npad,), N, jnp.int32)])
    col = jnp.concatenate([col, jnp.zeros((npad,), jnp.int32)])
    row = row.reshape(NS, NCHUNK, CH)
    col = col.reshape(NS, NCHUNK, CH)
    gidx = jnp.stack([2 * col, 2 * col + 1])
    fill = jnp.stack([jnp.zeros((CH, DH), jnp.float32),
                      jnp.ones((CH, DH), jnp.float32)])
    dummy_tbl = jnp.zeros((2 * N, DH), jnp.float32)

    deg = _sc_deg(dummy_tbl, gidx, row, fill)

    grid = (N // BN,)
    dis, t = pl.pallas_call(
        _tc_prologue,
        grid=grid,
        in_specs=[_half_block(), _row_block(D), _full_block(D, D)],
        out_specs=[_row_block(DH), _row_block(D)],
        out_shape=[jax.ShapeDtypeStruct((N, DH), jnp.float32),
                   jax.ShapeDtypeStruct((N, D), jnp.float32)],
    )(deg, x, convW[0])

    hs = []
    for i in range(NLAYERS):
        u = _sc_spmm(t.reshape(2 * N, DH), gidx, row, fill)
        if i == NLAYERS - 1:
            break
        h, t = pl.pallas_call(
            _tc_layer,
            grid=grid,
            in_specs=[_half_block(), _row_block(DH), _full_block(1, D),
                      _full_block(D, D)],
            out_specs=[_row_block(D), _row_block(D)],
            out_shape=[jax.ShapeDtypeStruct((N, D), jnp.float32),
                       jax.ShapeDtypeStruct((N, D), jnp.float32)],
        )(u, dis, convB[i].reshape(1, D), convW[i + 1])
        hs.append(h)

    out = pl.pallas_call(
        _tc_epilogue,
        grid=grid,
        in_specs=[_half_block(), _row_block(DH), _full_block(1, D)]
        + [_row_block(D)] * 5
        + [_full_block(NLAYERS * D, D), _full_block(1, D)],
        out_specs=_row_block(D),
        out_shape=jax.ShapeDtypeStruct((N, D), jnp.float32),
    )(u, dis, convB[NLAYERS - 1].reshape(1, D), *hs,
      lin_W, lin_b.reshape(1, D))
    return out
